# baseline (speedup 1.0000x reference)
import functools

import jax
import jax.numpy as jnp
from jax import lax
from jax.experimental import pallas as pl
from jax.experimental.pallas import tpu as pltpu
from jax.experimental.shard_map import shard_map
from jax.sharding import Mesh, PartitionSpec as P

T = 2048
D = 1024
E = 8
TOP_K = 2


def _moe_dense_body(x_ref, wg_ref, bg_ref, w_ref, b_ref,
                    out_ref, idx_ref, comb_ref, xb_ref, *, dn):
    n = pl.program_id(0)
    e = pl.program_id(1)

    @pl.when((n == 0) & (e == 0))
    def _gate():
        logits = jnp.dot(x_ref[...], wg_ref[...],
                         preferred_element_type=jnp.float32) + bg_ref[...]
        col = lax.broadcasted_iota(jnp.int32, (T, E), 1)
        m1 = jnp.max(logits, axis=1, keepdims=True)
        i1 = jnp.min(jnp.where(logits == m1, col, E), axis=1, keepdims=True)
        masked = jnp.where(col == i1, -jnp.inf, logits)
        m2 = jnp.max(masked, axis=1, keepdims=True)
        i2 = jnp.min(jnp.where(masked == m2, col, E), axis=1, keepdims=True)
        r = jnp.exp(m2 - m1)
        w1 = 1.0 / (1.0 + r)
        w2 = r / (1.0 + r)
        comb_ref[...] = (jnp.where(col == i1, w1, 0.0)
                         + jnp.where(col == i2, w2, 0.0))
        idx_ref[...] = jnp.concatenate([i1, i2], axis=1)
        xb_ref[...] = x_ref[...].astype(jnp.bfloat16)

    @pl.when(e == 0)
    def _init():
        out_ref[...] = jnp.dot(comb_ref[...], b_ref[...],
                               preferred_element_type=jnp.float32)

    ce = jnp.sum(
        comb_ref[...] * (lax.broadcasted_iota(jnp.int32, (T, E), 1) == e),
        axis=1, keepdims=True)
    acc = jnp.dot(xb_ref[...], w_ref[0].astype(jnp.bfloat16),
                  preferred_element_type=jnp.float32)
    out_ref[...] += ce * acc


def _moe_dense(x, Wg, bg2, W, b, nn):
    dloc = W.shape[2]
    dn = dloc // nn
    body = functools.partial(_moe_dense_body, dn=dn)
    out, idx = pl.pallas_call(
        body,
        grid=(nn, E),
        in_specs=[
            pl.BlockSpec((T, D), lambda n, e: (0, 0)),
            pl.BlockSpec((D, E), lambda n, e: (0, 0)),
            pl.BlockSpec((1, E), lambda n, e: (0, 0)),
            pl.BlockSpec((1, D, dn), lambda n, e: (e, 0, n)),
            pl.BlockSpec((E, dn), lambda n, e: (0, n)),
        ],
        out_specs=[
            pl.BlockSpec((T, dn), lambda n, e: (0, n)),
            pl.BlockSpec((T, TOP_K), lambda n, e: (0, 0)),
        ],
        out_shape=[
            jax.ShapeDtypeStruct((T, dloc), jnp.float32),
            jax.ShapeDtypeStruct((T, TOP_K), jnp.int32),
        ],
        scratch_shapes=[
            pltpu.VMEM((T, E), jnp.float32),
            pltpu.VMEM((T, D), jnp.bfloat16),
        ],
        compiler_params=pltpu.CompilerParams(
            dimension_semantics=("arbitrary", "arbitrary"),
        ),
    )(x, Wg, bg2, W, b)
    return out, idx


@jax.jit
def kernel(x, Wg, bg, W, b):
    bg2 = bg.reshape(1, E)
    devs = jax.devices()
    if len(devs) < 2:
        return _moe_dense(x, Wg, bg2, W, b, 2)
    mesh = Mesh(devs[:2], ("d",))
    f = shard_map(
        functools.partial(_moe_dense, nn=1),
        mesh=mesh,
        in_specs=(P(None, None), P(None, None), P(None, None),
                  P(None, None, "d"), P(None, "d")),
        out_specs=(P(None, "d"), P(None, None)),
        check_rep=False,
    )
    return f(x, Wg, bg2, W, b)


# speedup vs baseline: 8.6311x; 8.6311x over previous
import jax
import jax.numpy as jnp
from jax import lax
from jax.experimental import pallas as pl
from jax.experimental.pallas import tpu as pltpu

T = 2048
D = 1024
E = 8
TOP_K = 2
NN = 2
DN = D // NN
NB = 3
TOTAL = NN * E


def _w_copy(w_hbm, w_buf, sem, t, slot):
    n_ = t // E
    e_ = lax.rem(t, E)
    return pltpu.make_async_copy(
        w_hbm.at[e_, :, pl.ds(n_ * DN, DN)], w_buf.at[slot], sem.at[slot])


def _moe_dense_body(x_ref, wg_ref, bg_ref, w_hbm, b_ref,
                    out_ref, idx_ref, comb_ref, xb_ref, w_buf, sem):
    n = pl.program_id(0)
    e = pl.program_id(1)
    s = n * E + e

    @pl.when(s == 0)
    def _prologue():
        for k in range(2):
            _w_copy(w_hbm, w_buf, sem, k, k).start()

    slot = lax.rem(s, NB)
    t_next = s + 2

    @pl.when(t_next < TOTAL)
    def _prefetch():
        _w_copy(w_hbm, w_buf, sem, t_next, lax.rem(t_next, NB)).start()

    @pl.when((n == 0) & (e == 0))
    def _gate():
        logits = jnp.dot(x_ref[...], wg_ref[...],
                         preferred_element_type=jnp.float32) + bg_ref[...]
        col = lax.broadcasted_iota(jnp.int32, (T, E), 1)
        m1 = jnp.max(logits, axis=1, keepdims=True)
        i1 = jnp.min(jnp.where(logits == m1, col, E), axis=1, keepdims=True)
        masked = jnp.where(col == i1, -jnp.inf, logits)
        m2 = jnp.max(masked, axis=1, keepdims=True)
        i2 = jnp.min(jnp.where(masked == m2, col, E), axis=1, keepdims=True)
        r = jnp.exp(m2 - m1)
        w1 = 1.0 / (1.0 + r)
        w2 = r / (1.0 + r)
        comb_ref[...] = (jnp.where(col == i1, w1, 0.0)
                         + jnp.where(col == i2, w2, 0.0))
        idx_ref[...] = jnp.concatenate([i1, i2], axis=1)
        xb_ref[...] = x_ref[...].astype(jnp.bfloat16)

    @pl.when(e == 0)
    def _init():
        out_ref[...] = jnp.dot(comb_ref[...], b_ref[...],
                               preferred_element_type=jnp.float32)

    ce = jnp.sum(
        comb_ref[...] * (lax.broadcasted_iota(jnp.int32, (T, E), 1) == e),
        axis=1, keepdims=True)
    _w_copy(w_hbm, w_buf, sem, s, slot).wait()
    acc = jnp.dot(xb_ref[...], w_buf[slot].astype(jnp.bfloat16),
                  preferred_element_type=jnp.float32)
    out_ref[...] += ce * acc


@jax.jit
def kernel(x, Wg, bg, W, b):
    bg2 = bg.reshape(1, E)
    out, idx = pl.pallas_call(
        _moe_dense_body,
        grid=(NN, E),
        in_specs=[
            pl.BlockSpec((T, D), lambda n, e: (0, 0)),
            pl.BlockSpec((D, E), lambda n, e: (0, 0)),
            pl.BlockSpec((1, E), lambda n, e: (0, 0)),
            pl.BlockSpec(memory_space=pl.ANY),
            pl.BlockSpec((E, DN), lambda n, e: (0, n)),
        ],
        out_specs=[
            pl.BlockSpec((T, DN), lambda n, e: (0, n)),
            pl.BlockSpec((T, TOP_K), lambda n, e: (0, 0)),
        ],
        out_shape=[
            jax.ShapeDtypeStruct((T, D), jnp.float32),
            jax.ShapeDtypeStruct((T, TOP_K), jnp.int32),
        ],
        scratch_shapes=[
            pltpu.VMEM((T, E), jnp.float32),
            pltpu.VMEM((T, D), jnp.bfloat16),
            pltpu.VMEM((NB, D, DN), jnp.float32),
            pltpu.SemaphoreType.DMA((NB,)),
        ],
        compiler_params=pltpu.CompilerParams(
            dimension_semantics=("arbitrary", "arbitrary"),
        ),
    )(x, Wg, bg2, W, b)
    return out, idx


# speedup vs baseline: 9.0869x; 1.0528x over previous
import jax
import jax.numpy as jnp
from jax import lax
from jax.experimental import pallas as pl
from jax.experimental.pallas import tpu as pltpu

T = 2048
D = 1024
E = 8
TOP_K = 2
NN = 2
DN = D // NN


def _moe_dense_body(x_ref, wg_ref, bg_ref, w_ref, b_ref,
                    out_ref, idx_ref, comb_ref, xb_ref):
    n = pl.program_id(0)
    e = pl.program_id(1)

    @pl.when((n == 0) & (e == 0))
    def _gate():
        logits = jnp.dot(x_ref[...], wg_ref[...],
                         preferred_element_type=jnp.float32) + bg_ref[...]
        col = lax.broadcasted_iota(jnp.int32, (T, E), 1)
        m1 = jnp.max(logits, axis=1, keepdims=True)
        i1 = jnp.min(jnp.where(logits == m1, col, E), axis=1, keepdims=True)
        masked = jnp.where(col == i1, -jnp.inf, logits)
        m2 = jnp.max(masked, axis=1, keepdims=True)
        i2 = jnp.min(jnp.where(masked == m2, col, E), axis=1, keepdims=True)
        r = jnp.exp(m2 - m1)
        w1 = 1.0 / (1.0 + r)
        w2 = r / (1.0 + r)
        comb_ref[...] = (jnp.where(col == i1, w1, 0.0)
                         + jnp.where(col == i2, w2, 0.0))
        idx_ref[...] = jnp.concatenate([i1, i2], axis=1)
        xb_ref[...] = x_ref[...].astype(jnp.bfloat16)

    @pl.when(e == 0)
    def _init():
        out_ref[...] = jnp.dot(comb_ref[...], b_ref[...],
                               preferred_element_type=jnp.float32)

    ce = jnp.sum(
        comb_ref[...] * (lax.broadcasted_iota(jnp.int32, (T, E), 1) == e),
        axis=1, keepdims=True)
    acc = jnp.dot(xb_ref[...], w_ref[0].astype(jnp.bfloat16),
                  preferred_element_type=jnp.float32)
    out_ref[...] += ce * acc


@jax.jit
def kernel(x, Wg, bg, W, b):
    bg2 = bg.reshape(1, E)
    out, idx = pl.pallas_call(
        _moe_dense_body,
        grid=(NN, E),
        in_specs=[
            pl.BlockSpec((T, D), lambda n, e: (0, 0)),
            pl.BlockSpec((D, E), lambda n, e: (0, 0)),
            pl.BlockSpec((1, E), lambda n, e: (0, 0)),
            pl.BlockSpec((1, D, DN), lambda n, e: (e, 0, n)),
            pl.BlockSpec((E, DN), lambda n, e: (0, n)),
        ],
        out_specs=[
            pl.BlockSpec((T, DN), lambda n, e: (0, n)),
            pl.BlockSpec((T, TOP_K), lambda n, e: (0, 0)),
        ],
        out_shape=[
            jax.ShapeDtypeStruct((T, D), jnp.float32),
            jax.ShapeDtypeStruct((T, TOP_K), jnp.int32),
        ],
        scratch_shapes=[
            pltpu.VMEM((T, E), jnp.float32),
            pltpu.VMEM((T, D), jnp.bfloat16),
        ],
        compiler_params=pltpu.CompilerParams(
            dimension_semantics=("arbitrary", "arbitrary"),
        ),
    )(x, Wg, bg2, W, b)
    return out, idx


# speedup vs baseline: 9.1041x; 1.0019x over previous
import jax
import jax.numpy as jnp
from jax import lax
from jax.experimental import pallas as pl
from jax.experimental.pallas import tpu as pltpu

T = 2048
D = 1024
E = 8
TOP_K = 2
NN = 2
DN = D // NN
NB = 3


def _w_copy(w_hbm, w_buf, sem, n, e):
    slot = e % NB
    return pltpu.make_async_copy(
        w_hbm.at[e, :, pl.ds(n * DN, DN)], w_buf.at[slot], sem.at[slot])


def _moe_dense_body(x_ref, wg_ref, bg_ref, w_hbm, b_ref,
                    out_ref, idx_ref, comb_ref, xb_ref, w_buf, sem):
    n = pl.program_id(0)

    for k in range(2):
        _w_copy(w_hbm, w_buf, sem, n, k).start()

    @pl.when(n == 0)
    def _gate():
        logits = jnp.dot(x_ref[...], wg_ref[...],
                         preferred_element_type=jnp.float32) + bg_ref[...]
        col = lax.broadcasted_iota(jnp.int32, (T, E), 1)
        m1 = jnp.max(logits, axis=1, keepdims=True)
        i1 = jnp.min(jnp.where(logits == m1, col, E), axis=1, keepdims=True)
        masked = jnp.where(col == i1, -jnp.inf, logits)
        m2 = jnp.max(masked, axis=1, keepdims=True)
        i2 = jnp.min(jnp.where(masked == m2, col, E), axis=1, keepdims=True)
        r = jnp.exp(m2 - m1)
        w1 = 1.0 / (1.0 + r)
        w2 = r / (1.0 + r)
        comb_ref[...] = (jnp.where(col == i1, w1, 0.0)
                         + jnp.where(col == i2, w2, 0.0))
        idx_ref[...] = jnp.concatenate([i1, i2], axis=1)
        xb_ref[...] = x_ref[...].astype(jnp.bfloat16)

    comb = comb_ref[...]
    col = lax.broadcasted_iota(jnp.int32, (T, E), 1)
    acc_out = jnp.dot(comb, b_ref[...], preferred_element_type=jnp.float32)
    for e in range(E):
        if e + 2 < E:
            _w_copy(w_hbm, w_buf, sem, n, e + 2).start()
        _w_copy(w_hbm, w_buf, sem, n, e).wait()
        ce = jnp.sum(comb * (col == e), axis=1, keepdims=True)
        acc = jnp.dot(xb_ref[...], w_buf[e % NB].astype(jnp.bfloat16),
                      preferred_element_type=jnp.float32)
        acc_out = acc_out + ce * acc
    out_ref[...] = acc_out


@jax.jit
def kernel(x, Wg, bg, W, b):
    bg2 = bg.reshape(1, E)
    out, idx = pl.pallas_call(
        _moe_dense_body,
        grid=(NN,),
        in_specs=[
            pl.BlockSpec((T, D), lambda n: (0, 0)),
            pl.BlockSpec((D, E), lambda n: (0, 0)),
            pl.BlockSpec((1, E), lambda n: (0, 0)),
            pl.BlockSpec(memory_space=pl.ANY),
            pl.BlockSpec((E, DN), lambda n: (0, n)),
        ],
        out_specs=[
            pl.BlockSpec((T, DN), lambda n: (0, n)),
            pl.BlockSpec((T, TOP_K), lambda n: (0, 0)),
        ],
        out_shape=[
            jax.ShapeDtypeStruct((T, D), jnp.float32),
            jax.ShapeDtypeStruct((T, TOP_K), jnp.int32),
        ],
        scratch_shapes=[
            pltpu.VMEM((T, E), jnp.float32),
            pltpu.VMEM((T, D), jnp.bfloat16),
            pltpu.VMEM((NB, D, DN), jnp.float32),
            pltpu.SemaphoreType.DMA((NB,)),
        ],
        compiler_params=pltpu.CompilerParams(
            dimension_semantics=("arbitrary",),
        ),
    )(x, Wg, bg2, W, b)
    return out, idx
